# baseline (speedup 1.0000x reference)
import functools

import jax
import jax.numpy as jnp
from jax import lax
from jax.experimental import pallas as pl
from jax.experimental.pallas import tpu as pltpu
from jax.experimental.pallas import tpu_sc as plsc

B = 1024
T = 200
VOCAB = 100000
N_EMBD = 128
COUNT_DIM = 8
NUM_PROPS = 520
FP_DIM = NUM_PROPS - COUNT_DIM
D_TOT = NUM_PROPS + T

NC, NS = 2, 16
NW = NC * NS
BPW = B // NW
CB = 1
CH = CB * T
NCH = BPW // CB
L = 16
NCHK = N_EMBD // L
HALF = NCHK // 2


def _rotate_row(buf, t, C, S, tt):
    g = [buf[t, pl.ds(L * c, L)] for c in range(NCHK)]
    for c in range(HALF):
        buf[t, pl.ds(L * c, L)] = g[c] * C[c] - g[c + HALF] * S[c] + tt[c]
        buf[t, pl.ds(L * (c + HALF), L)] = (g[c + HALF] * C[c]
                                            + g[c] * S[c] + tt[c + HALF])


def _sc_gather_rotary(table, idx, cosh, sinh, tt1):
    mesh = plsc.VectorSubcoreMesh(core_axis_name="c", subcore_axis_name="s")

    @functools.partial(
        pl.kernel,
        mesh=mesh,
        out_type=jax.ShapeDtypeStruct((B, D_TOT, N_EMBD), jnp.float32),
        scratch_types=[
            pltpu.VMEM((CH,), jnp.int32),
            pltpu.VMEM((CH,), jnp.int32),
            pltpu.VMEM((CH,), jnp.int32),
            pltpu.VMEM((CH, N_EMBD), jnp.float32),
            pltpu.VMEM((CH, N_EMBD), jnp.float32),
            pltpu.VMEM((CH, N_EMBD), jnp.float32),
            pltpu.VMEM((T, N_EMBD // 2), jnp.float32),
            pltpu.VMEM((T, N_EMBD // 2), jnp.float32),
            pltpu.VMEM((N_EMBD,), jnp.float32),
            pltpu.SemaphoreType.DMA,
            pltpu.SemaphoreType.DMA,
            pltpu.SemaphoreType.DMA,
            pltpu.SemaphoreType.DMA,
            pltpu.SemaphoreType.DMA,
            pltpu.SemaphoreType.DMA,
        ],
    )
    def k(table_hbm, idx_hbm, cos_hbm, sin_hbm, tt_hbm, out_hbm,
          idx0, idx1, idx2, buf0, buf1, buf2, cos_v, sin_v, tt_v,
          gs0, gs1, gs2, ss0, ss1, ss2):
        wid = lax.axis_index("s") * NC + lax.axis_index("c")
        b0 = wid * BPW
        pltpu.sync_copy(cos_hbm, cos_v)
        pltpu.sync_copy(sin_hbm, sin_v)
        pltpu.sync_copy(tt_hbm, tt_v)
        tt = [tt_v[pl.ds(L * c, L)] for c in range(NCHK)]
        idxs, bufs = (idx0, idx1, idx2), (buf0, buf1, buf2)
        gsem, ssem = (gs0, gs1, gs2), (ss0, ss1, ss2)

        def fetch(i):
            p = i % 3
            pltpu.sync_copy(idx_hbm.at[pl.ds((b0 + i * CB) * T, CH)], idxs[p])
            return pltpu.async_copy(table_hbm.at[idxs[p]], bufs[p], gsem[p])

        def put(i):
            return pltpu.async_copy(
                bufs[i % 3], out_hbm.at[b0 + i, pl.ds(NUM_PROPS, T)],
                ssem[i % 3])

        gh = [None] * NCH
        sh = [None] * NCH
        gh[0] = fetch(0)
        gh[1] = fetch(1)
        for j in range(NCH // 2):
            i0, i1 = 2 * j, 2 * j + 1
            if i0 + 2 < NCH:
                if j >= 1:
                    sh[i0 - 1].wait()
                gh[i0 + 2] = fetch(i0 + 2)
            gh[i0].wait()
            gh[i1].wait()

            @plsc.parallel_loop(0, T, unroll=2)
            def _(t, ba=bufs[i0 % 3], bb=bufs[i1 % 3]):
                C = [cos_v[t, pl.ds(L * c, L)] for c in range(HALF)]
                S = [sin_v[t, pl.ds(L * c, L)] for c in range(HALF)]
                _rotate_row(ba, t, C, S, tt)
                _rotate_row(bb, t, C, S, tt)
            sh[i0] = put(i0)
            sh[i1] = put(i1)
            if i1 + 2 < NCH:
                sh[i0].wait()
                gh[i1 + 2] = fetch(i1 + 2)
        sh[NCH - 2].wait()
        sh[NCH - 1].wait()

    return k(table, idx, cosh, sinh, tt1)


BB = 16


def _prop_body(prop_ref, base_ref, diff_ref, o_ref, out_ref):
    del o_ref
    propf = prop_ref[...].astype(jnp.float32)
    out_ref[...] = (base_ref[...][None]
                    + propf[:, :, None] * diff_ref[...][None])


def kernel(token, prop, tok_table, type_table, prop_type_table, cnt_bit,
           cnt_val, fp_pair, fp_bit, fp_val):
    idx = token.reshape(B * T).astype(jnp.int32)

    inv_freq = 1.0 / (10000.0 ** (jnp.arange(0, N_EMBD, 2, dtype=jnp.float32)
                                  / N_EMBD))
    freqs = jnp.arange(T, dtype=jnp.float32)[:, None] * inv_freq[None, :]
    cosh, sinh = jnp.cos(freqs), jnp.sin(freqs)
    tt1 = type_table[1]

    o0 = _sc_gather_rotary(tok_table, idx, cosh, sinh, tt1)

    base_cnt = cnt_val[0][None] + cnt_bit + prop_type_table[0][None]
    pair_rep = jnp.repeat(fp_pair, 2, axis=0)
    bit_rep = jnp.tile(fp_bit, (FP_DIM // 2, 1))
    base_fp = fp_val[0][None] + pair_rep + bit_rep + prop_type_table[1][None]
    base = jnp.concatenate([base_cnt, base_fp], axis=0) + type_table[0][None]
    diff = jnp.concatenate([
        jnp.broadcast_to(cnt_val[1] - cnt_val[0], (COUNT_DIM, N_EMBD)),
        jnp.broadcast_to(fp_val[1] - fp_val[0], (FP_DIM, N_EMBD)),
    ], axis=0)

    return pl.pallas_call(
        _prop_body,
        grid=(B // BB,),
        in_specs=[
            pl.BlockSpec((BB, NUM_PROPS), lambda i: (i, 0)),
            pl.BlockSpec((NUM_PROPS, N_EMBD), lambda i: (0, 0)),
            pl.BlockSpec((NUM_PROPS, N_EMBD), lambda i: (0, 0)),
            pl.BlockSpec(memory_space=pl.ANY),
        ],
        out_specs=pl.BlockSpec((BB, NUM_PROPS, N_EMBD), lambda i: (i, 0, 0)),
        out_shape=jax.ShapeDtypeStruct((B, D_TOT, N_EMBD), jnp.float32),
        input_output_aliases={3: 0},
    )(prop, base, diff, o0)


# speedup vs baseline: 1.0858x; 1.0858x over previous
import functools

import jax
import jax.numpy as jnp
from jax import lax
from jax.experimental import pallas as pl
from jax.experimental.pallas import tpu as pltpu
from jax.experimental.pallas import tpu_sc as plsc

B = 1024
T = 200
VOCAB = 100000
N_EMBD = 128
COUNT_DIM = 8
NUM_PROPS = 520
FP_DIM = NUM_PROPS - COUNT_DIM
D_TOT = NUM_PROPS + T

NC, NS = 2, 16
NW = NC * NS
BPW = B // NW
CB = 1
CH = CB * T
NCH = BPW // CB
L = 16
NCHK = N_EMBD // L
HALF = NCHK // 2


def _rotate_row(buf, t, C, S, tt):
    g = [buf[t, pl.ds(L * c, L)] for c in range(NCHK)]
    for c in range(HALF):
        buf[t, pl.ds(L * c, L)] = g[c] * C[c] - g[c + HALF] * S[c] + tt[c]
        buf[t, pl.ds(L * (c + HALF), L)] = (g[c + HALF] * C[c]
                                            + g[c] * S[c] + tt[c + HALF])


def _sc_gather_rotary(table, idx, cosh, sinh, tt1):
    mesh = plsc.VectorSubcoreMesh(core_axis_name="c", subcore_axis_name="s")

    @functools.partial(
        pl.kernel,
        mesh=mesh,
        out_type=jax.ShapeDtypeStruct((B * D_TOT, N_EMBD), jnp.float32),
        scratch_types=[
            pltpu.VMEM((BPW * T,), jnp.int32),
            pltpu.VMEM((CH, N_EMBD), jnp.float32),
            pltpu.VMEM((CH, N_EMBD), jnp.float32),
            pltpu.VMEM((T, N_EMBD // 2), jnp.float32),
            pltpu.VMEM((T, N_EMBD // 2), jnp.float32),
            pltpu.VMEM((N_EMBD,), jnp.float32),
            pltpu.SemaphoreType.DMA,
            pltpu.SemaphoreType.DMA,
            pltpu.SemaphoreType.DMA,
            pltpu.SemaphoreType.DMA,
        ],
    )
    def k(table_hbm, idx_hbm, cos_hbm, sin_hbm, tt_hbm, out_hbm,
          idx_v, buf0, buf1, cos_v, sin_v, tt_v,
          gs0, gs1, ss0, ss1):
        wid = lax.axis_index("s") * NC + lax.axis_index("c")
        b0 = wid * BPW
        pltpu.sync_copy(idx_hbm.at[pl.ds(b0 * T, BPW * T)], idx_v)
        pltpu.sync_copy(cos_hbm, cos_v)
        pltpu.sync_copy(sin_hbm, sin_v)
        pltpu.sync_copy(tt_hbm, tt_v)
        tt = [tt_v[pl.ds(L * c, L)] for c in range(NCHK)]
        bufs = (buf0, buf1)
        gsem, ssem = (gs0, gs1), (ss0, ss1)

        def fetch(i):
            p = i % 2
            return pltpu.async_copy(
                table_hbm.at[idx_v.at[pl.ds(i * CH, CH)]], bufs[p], gsem[p])

        def put(i):
            return pltpu.async_copy(
                bufs[i % 2],
                out_hbm.at[pl.ds((b0 + i) * D_TOT + NUM_PROPS, T)],
                ssem[i % 2])

        gh = [None] * NCH
        sh = [None] * NCH
        gh[0] = fetch(0)
        for i in range(NCH):
            if i + 1 < NCH:
                if i >= 1:
                    sh[i - 1].wait()
                gh[i + 1] = fetch(i + 1)
            gh[i].wait()

            @plsc.parallel_loop(0, T, unroll=2)
            def _(t, buf=bufs[i % 2]):
                C = [cos_v[t, pl.ds(L * c, L)] for c in range(HALF)]
                S = [sin_v[t, pl.ds(L * c, L)] for c in range(HALF)]
                _rotate_row(buf, t, C, S, tt)

            sh[i] = put(i)
        sh[NCH - 2].wait()
        sh[NCH - 1].wait()

    return k(table, idx, cosh, sinh, tt1)


BB = 16


def _prop_body(prop_ref, base_ref, diff_ref, o_ref, out_ref):
    del o_ref
    propf = prop_ref[...].astype(jnp.float32)
    out_ref[...] = (base_ref[...][None]
                    + propf[:, :, None] * diff_ref[...][None])


def kernel(token, prop, tok_table, type_table, prop_type_table, cnt_bit,
           cnt_val, fp_pair, fp_bit, fp_val):
    idx = token.reshape(B * T).astype(jnp.int32)

    inv_freq = 1.0 / (10000.0 ** (jnp.arange(0, N_EMBD, 2, dtype=jnp.float32)
                                  / N_EMBD))
    freqs = jnp.arange(T, dtype=jnp.float32)[:, None] * inv_freq[None, :]
    cosh, sinh = jnp.cos(freqs), jnp.sin(freqs)
    tt1 = type_table[1]

    o0 = _sc_gather_rotary(tok_table, idx, cosh, sinh, tt1)
    o0 = o0.reshape(B, D_TOT, N_EMBD)

    base_cnt = cnt_val[0][None] + cnt_bit + prop_type_table[0][None]
    pair_rep = jnp.repeat(fp_pair, 2, axis=0)
    bit_rep = jnp.tile(fp_bit, (FP_DIM // 2, 1))
    base_fp = fp_val[0][None] + pair_rep + bit_rep + prop_type_table[1][None]
    base = jnp.concatenate([base_cnt, base_fp], axis=0) + type_table[0][None]
    diff = jnp.concatenate([
        jnp.broadcast_to(cnt_val[1] - cnt_val[0], (COUNT_DIM, N_EMBD)),
        jnp.broadcast_to(fp_val[1] - fp_val[0], (FP_DIM, N_EMBD)),
    ], axis=0)

    return pl.pallas_call(
        _prop_body,
        grid=(B // BB,),
        in_specs=[
            pl.BlockSpec((BB, NUM_PROPS), lambda i: (i, 0)),
            pl.BlockSpec((NUM_PROPS, N_EMBD), lambda i: (0, 0)),
            pl.BlockSpec((NUM_PROPS, N_EMBD), lambda i: (0, 0)),
            pl.BlockSpec(memory_space=pl.ANY),
        ],
        out_specs=pl.BlockSpec((BB, NUM_PROPS, N_EMBD), lambda i: (i, 0, 0)),
        out_shape=jax.ShapeDtypeStruct((B, D_TOT, N_EMBD), jnp.float32),
        input_output_aliases={3: 0},
    )(prop, base, diff, o0)


# speedup vs baseline: 1.1415x; 1.0513x over previous
import functools

import jax
import jax.numpy as jnp
from jax import lax
from jax.experimental import pallas as pl
from jax.experimental.pallas import tpu as pltpu
from jax.experimental.pallas import tpu_sc as plsc

B = 1024
T = 200
VOCAB = 100000
N_EMBD = 128
COUNT_DIM = 8
NUM_PROPS = 520
FP_DIM = NUM_PROPS - COUNT_DIM
D_TOT = NUM_PROPS + T

NC, NS = 2, 16
NW = NC * NS
BPW = B // NW
CB = 1
CH = CB * T
NCH = BPW // CB
L = 16
NCHK = N_EMBD // L
HALF = NCHK // 2


def _rotate_row(buf, t, C, S, tt):
    g = [buf[t, pl.ds(L * c, L)] for c in range(NCHK)]
    for c in range(HALF):
        buf[t, pl.ds(L * c, L)] = g[c] * C[c] - g[c + HALF] * S[c] + tt[c]
        buf[t, pl.ds(L * (c + HALF), L)] = (g[c + HALF] * C[c]
                                            + g[c] * S[c] + tt[c + HALF])


def _sc_gather_rotary(table, idx, cosh, sinh, tt1):
    mesh = plsc.VectorSubcoreMesh(core_axis_name="c", subcore_axis_name="s")

    @functools.partial(
        pl.kernel,
        mesh=mesh,
        out_type=jax.ShapeDtypeStruct((B * D_TOT, N_EMBD), jnp.float32),
        scratch_types=[
            pltpu.VMEM((BPW * T,), jnp.int32),
            pltpu.VMEM((CH, N_EMBD), jnp.float32),
            pltpu.VMEM((CH, N_EMBD), jnp.float32),
            pltpu.VMEM((T, N_EMBD // 2), jnp.float32),
            pltpu.VMEM((T, N_EMBD // 2), jnp.float32),
            pltpu.VMEM((N_EMBD,), jnp.float32),
            pltpu.SemaphoreType.DMA,
            pltpu.SemaphoreType.DMA,
            pltpu.SemaphoreType.DMA,
            pltpu.SemaphoreType.DMA,
        ],
    )
    def k(table_hbm, idx_hbm, cos_hbm, sin_hbm, tt_hbm, out_hbm,
          idx_v, buf0, buf1, cos_v, sin_v, tt_v,
          gs0, gs1, ss0, ss1):
        wid = lax.axis_index("s") * NC + lax.axis_index("c")
        b0 = wid * BPW
        pltpu.sync_copy(idx_hbm.at[pl.ds(b0 * T, BPW * T)], idx_v)
        pltpu.sync_copy(cos_hbm, cos_v)
        pltpu.sync_copy(sin_hbm, sin_v)
        pltpu.sync_copy(tt_hbm, tt_v)
        tt = [tt_v[pl.ds(L * c, L)] for c in range(NCHK)]
        bufs = (buf0, buf1)
        gsem, ssem = (gs0, gs1), (ss0, ss1)

        def fetch(i):
            p = i % 2
            return pltpu.async_copy(
                table_hbm.at[idx_v.at[pl.ds(i * CH, CH)]], bufs[p], gsem[p])

        def put(i):
            return pltpu.async_copy(
                bufs[i % 2],
                out_hbm.at[pl.ds((b0 + i) * D_TOT + NUM_PROPS, T)],
                ssem[i % 2])

        gh = [None] * NCH
        sh = [None] * NCH
        gh[0] = fetch(0)
        for i in range(NCH):
            if i + 1 < NCH:
                if i >= 1:
                    sh[i - 1].wait()
                gh[i + 1] = fetch(i + 1)
            gh[i].wait()

            @plsc.parallel_loop(0, T, unroll=2)
            def _(t, buf=bufs[i % 2]):
                C = [cos_v[t, pl.ds(L * c, L)] for c in range(HALF)]
                S = [sin_v[t, pl.ds(L * c, L)] for c in range(HALF)]
                _rotate_row(buf, t, C, S, tt)

            sh[i] = put(i)
        sh[NCH - 2].wait()
        sh[NCH - 1].wait()

    return k(table, idx, cosh, sinh, tt1)


BB = 32


def _prop_body(prop_ref, base_ref, diff_ref, o_ref, out_ref):
    del o_ref
    propf = prop_ref[...].astype(jnp.float32)
    out_ref[...] = (base_ref[...][None]
                    + propf[:, :, None] * diff_ref[...][None])


def kernel(token, prop, tok_table, type_table, prop_type_table, cnt_bit,
           cnt_val, fp_pair, fp_bit, fp_val):
    idx = token.reshape(B * T).astype(jnp.int32)

    inv_freq = 1.0 / (10000.0 ** (jnp.arange(0, N_EMBD, 2, dtype=jnp.float32)
                                  / N_EMBD))
    freqs = jnp.arange(T, dtype=jnp.float32)[:, None] * inv_freq[None, :]
    cosh, sinh = jnp.cos(freqs), jnp.sin(freqs)
    tt1 = type_table[1]

    o0 = _sc_gather_rotary(tok_table, idx, cosh, sinh, tt1)
    o0 = o0.reshape(B, D_TOT, N_EMBD)

    base_cnt = cnt_val[0][None] + cnt_bit + prop_type_table[0][None]
    pair_rep = jnp.repeat(fp_pair, 2, axis=0)
    bit_rep = jnp.tile(fp_bit, (FP_DIM // 2, 1))
    base_fp = fp_val[0][None] + pair_rep + bit_rep + prop_type_table[1][None]
    base = jnp.concatenate([base_cnt, base_fp], axis=0) + type_table[0][None]
    diff = jnp.concatenate([
        jnp.broadcast_to(cnt_val[1] - cnt_val[0], (COUNT_DIM, N_EMBD)),
        jnp.broadcast_to(fp_val[1] - fp_val[0], (FP_DIM, N_EMBD)),
    ], axis=0)

    return pl.pallas_call(
        _prop_body,
        grid=(B // BB,),
        in_specs=[
            pl.BlockSpec((BB, NUM_PROPS), lambda i: (i, 0)),
            pl.BlockSpec((NUM_PROPS, N_EMBD), lambda i: (0, 0)),
            pl.BlockSpec((NUM_PROPS, N_EMBD), lambda i: (0, 0)),
            pl.BlockSpec(memory_space=pl.ANY),
        ],
        out_specs=pl.BlockSpec((BB, NUM_PROPS, N_EMBD), lambda i: (i, 0, 0)),
        out_shape=jax.ShapeDtypeStruct((B, D_TOT, N_EMBD), jnp.float32),
        input_output_aliases={3: 0},
    )(prop, base, diff, o0)


# speedup vs baseline: 1.1557x; 1.0125x over previous
import functools

import jax
import jax.numpy as jnp
from jax import lax
from jax.experimental import pallas as pl
from jax.experimental.pallas import tpu as pltpu
from jax.experimental.pallas import tpu_sc as plsc

B = 1024
T = 200
VOCAB = 100000
N_EMBD = 128
COUNT_DIM = 8
NUM_PROPS = 520
FP_DIM = NUM_PROPS - COUNT_DIM
D_TOT = NUM_PROPS + T

NC, NS = 2, 16
NW = NC * NS
BPW = B // NW
CB = 1
CH = CB * T
NCH = BPW // CB
L = 16
NCHK = N_EMBD // L
HALF = NCHK // 2


def _rotate_row(buf, t, C, S, tt):
    g = [buf[t, pl.ds(L * c, L)] for c in range(NCHK)]
    for c in range(HALF):
        buf[t, pl.ds(L * c, L)] = g[c] * C[c] - g[c + HALF] * S[c] + tt[c]
        buf[t, pl.ds(L * (c + HALF), L)] = (g[c + HALF] * C[c]
                                            + g[c] * S[c] + tt[c + HALF])


def _sc_gather_rotary(table, idx, cosh, sinh, tt1):
    mesh = plsc.VectorSubcoreMesh(core_axis_name="c", subcore_axis_name="s")

    @functools.partial(
        pl.kernel,
        mesh=mesh,
        out_type=jax.ShapeDtypeStruct((B * D_TOT, N_EMBD), jnp.float32),
        scratch_types=[
            pltpu.VMEM((BPW * T,), jnp.int32),
            pltpu.VMEM((CH, N_EMBD), jnp.float32),
            pltpu.VMEM((CH, N_EMBD), jnp.float32),
            pltpu.VMEM((T, N_EMBD // 2), jnp.float32),
            pltpu.VMEM((T, N_EMBD // 2), jnp.float32),
            pltpu.VMEM((N_EMBD,), jnp.float32),
            pltpu.SemaphoreType.DMA,
            pltpu.SemaphoreType.DMA,
            pltpu.SemaphoreType.DMA,
            pltpu.SemaphoreType.DMA,
        ],
    )
    def k(table_hbm, idx_hbm, cos_hbm, sin_hbm, tt_hbm, out_hbm,
          idx_v, buf0, buf1, cos_v, sin_v, tt_v,
          gs0, gs1, ss0, ss1):
        wid = lax.axis_index("s") * NC + lax.axis_index("c")
        b0 = wid * BPW
        pltpu.sync_copy(idx_hbm.at[pl.ds(b0 * T, BPW * T)], idx_v)
        pltpu.sync_copy(cos_hbm, cos_v)
        pltpu.sync_copy(sin_hbm, sin_v)
        pltpu.sync_copy(tt_hbm, tt_v)
        tt = [tt_v[pl.ds(L * c, L)] for c in range(NCHK)]
        bufs = (buf0, buf1)
        gsem, ssem = (gs0, gs1), (ss0, ss1)

        def fetch(i):
            p = i % 2
            return pltpu.async_copy(
                table_hbm.at[idx_v.at[pl.ds(i * CH, CH)]], bufs[p], gsem[p])

        def put(i):
            return pltpu.async_copy(
                bufs[i % 2],
                out_hbm.at[pl.ds((b0 + i) * D_TOT + NUM_PROPS, T)],
                ssem[i % 2])

        gh = [None] * NCH
        sh = [None] * NCH
        gh[0] = fetch(0)
        for i in range(NCH):
            if i + 1 < NCH:
                if i >= 1:
                    sh[i - 1].wait()
                gh[i + 1] = fetch(i + 1)
            gh[i].wait()

            @plsc.parallel_loop(0, T, unroll=2)
            def _(t, buf=bufs[i % 2]):
                C = [cos_v[t, pl.ds(L * c, L)] for c in range(HALF)]
                S = [sin_v[t, pl.ds(L * c, L)] for c in range(HALF)]
                _rotate_row(buf, t, C, S, tt)

            sh[i] = put(i)
        sh[NCH - 2].wait()
        sh[NCH - 1].wait()

    return k(table, idx, cosh, sinh, tt1)


BB = 64


def _prop_body(prop_ref, base_ref, diff_ref, o_ref, out_ref):
    del o_ref
    propf = prop_ref[...].astype(jnp.float32)
    out_ref[...] = (base_ref[...][None]
                    + propf[:, :, None] * diff_ref[...][None])


def kernel(token, prop, tok_table, type_table, prop_type_table, cnt_bit,
           cnt_val, fp_pair, fp_bit, fp_val):
    idx = token.reshape(B * T).astype(jnp.int32)

    inv_freq = 1.0 / (10000.0 ** (jnp.arange(0, N_EMBD, 2, dtype=jnp.float32)
                                  / N_EMBD))
    freqs = jnp.arange(T, dtype=jnp.float32)[:, None] * inv_freq[None, :]
    cosh, sinh = jnp.cos(freqs), jnp.sin(freqs)
    tt1 = type_table[1]

    o0 = _sc_gather_rotary(tok_table, idx, cosh, sinh, tt1)
    o0 = o0.reshape(B, D_TOT, N_EMBD)

    base_cnt = cnt_val[0][None] + cnt_bit + prop_type_table[0][None]
    pair_rep = jnp.repeat(fp_pair, 2, axis=0)
    bit_rep = jnp.tile(fp_bit, (FP_DIM // 2, 1))
    base_fp = fp_val[0][None] + pair_rep + bit_rep + prop_type_table[1][None]
    base = jnp.concatenate([base_cnt, base_fp], axis=0) + type_table[0][None]
    diff = jnp.concatenate([
        jnp.broadcast_to(cnt_val[1] - cnt_val[0], (COUNT_DIM, N_EMBD)),
        jnp.broadcast_to(fp_val[1] - fp_val[0], (FP_DIM, N_EMBD)),
    ], axis=0)

    return pl.pallas_call(
        _prop_body,
        grid=(B // BB,),
        in_specs=[
            pl.BlockSpec((BB, NUM_PROPS), lambda i: (i, 0)),
            pl.BlockSpec((NUM_PROPS, N_EMBD), lambda i: (0, 0)),
            pl.BlockSpec((NUM_PROPS, N_EMBD), lambda i: (0, 0)),
            pl.BlockSpec(memory_space=pl.ANY),
        ],
        out_specs=pl.BlockSpec((BB, NUM_PROPS, N_EMBD), lambda i: (i, 0, 0)),
        out_shape=jax.ShapeDtypeStruct((B, D_TOT, N_EMBD), jnp.float32),
        input_output_aliases={3: 0},
    )(prop, base, diff, o0)
